# baseline (speedup 1.0000x reference)
import jax
import jax.numpy as jnp
from jax.experimental import pallas as pl

_NUM_TOKENS = 32768
_INPUT_SIZE = 768
_NUM_EXPERTS = 8
_BALANCE_LOSS_WEIGHT = 0.1
_BLOCK = 1024


def _gate_kernel(x_ref, w1_ref, w2_ref,
                 idx_ref, score_ref, loss_ref, load_ref, imp_ref):
    i = pl.program_id(0)
    nsteps = pl.num_programs(0)

    x = x_ref[...]
    w1 = w1_ref[...]
    w2 = w2_ref[...]

    ht = jnp.tanh(jax.lax.dot_general(
        w1, x, (((1,), (1,)), ((), ())),
        preferred_element_type=jnp.float32))
    logits = jax.lax.dot_general(
        w2, ht, (((1,), (0,)), ((), ())),
        preferred_element_type=jnp.float32)

    m = jnp.max(logits, axis=0, keepdims=True)
    iota = jax.lax.broadcasted_iota(jnp.int32, logits.shape, 0)
    idx = jnp.min(jnp.where(logits == m, iota, _NUM_EXPERTS), axis=0)
    idx_ref[...] = idx
    score_ref[...] = m[0]

    e = jnp.exp(logits - m)
    scores = e / jnp.sum(e, axis=0, keepdims=True)
    imp_part = jnp.sum(scores, axis=1, keepdims=True)

    onehot = (iota == idx[None, :]).astype(jnp.float32)
    load_part = jnp.sum(onehot, axis=1, keepdims=True)

    @pl.when(i == 0)
    def _init():
        imp_ref[...] = jnp.zeros_like(imp_ref)
        load_ref[...] = jnp.zeros_like(load_ref)

    imp_ref[...] += imp_part
    load_ref[...] += load_part

    @pl.when(i == nsteps - 1)
    def _finalize():
        inv_n = 1.0 / _NUM_TOKENS
        imp_mean = imp_ref[...] * inv_n
        load_mean = load_ref[...] * inv_n
        imp_ref[...] = imp_mean
        load_ref[...] = load_mean
        loss_ref[...] = (_NUM_EXPERTS * _BALANCE_LOSS_WEIGHT) * jnp.sum(
            imp_mean * load_mean, axis=0, keepdims=True)


def kernel(x, W1, W2):
    n_tokens = x.shape[0]
    grid = (n_tokens // _BLOCK,)
    idx, score, loss, load_mean, imp_mean = pl.pallas_call(
        _gate_kernel,
        grid=grid,
        in_specs=[
            pl.BlockSpec((_BLOCK, _INPUT_SIZE), lambda i: (i, 0)),
            pl.BlockSpec((_NUM_EXPERTS, _INPUT_SIZE), lambda i: (0, 0)),
            pl.BlockSpec((_NUM_EXPERTS, _NUM_EXPERTS), lambda i: (0, 0)),
        ],
        out_specs=[
            pl.BlockSpec((_BLOCK,), lambda i: (i,)),
            pl.BlockSpec((_BLOCK,), lambda i: (i,)),
            pl.BlockSpec((1, 1), lambda i: (0, 0)),
            pl.BlockSpec((_NUM_EXPERTS, 1), lambda i: (0, 0)),
            pl.BlockSpec((_NUM_EXPERTS, 1), lambda i: (0, 0)),
        ],
        out_shape=[
            jax.ShapeDtypeStruct((n_tokens,), jnp.int32),
            jax.ShapeDtypeStruct((n_tokens,), jnp.float32),
            jax.ShapeDtypeStruct((1, 1), jnp.float32),
            jax.ShapeDtypeStruct((_NUM_EXPERTS, 1), jnp.float32),
            jax.ShapeDtypeStruct((_NUM_EXPERTS, 1), jnp.float32),
        ],
    )(x, W1, W2)
    return (idx, score, loss[0, 0], load_mean[:, 0], imp_mean[:, 0])


# speedup vs baseline: 1.3232x; 1.3232x over previous
import jax
import jax.numpy as jnp
from jax.experimental import pallas as pl

_NUM_TOKENS = 32768
_INPUT_SIZE = 768
_NUM_EXPERTS = 8
_BALANCE_LOSS_WEIGHT = 0.1
_BLOCK = 4096
_HALF = _INPUT_SIZE // 2


def _gate_kernel(xa_ref, xb_ref, w1_ref, w2_ref,
                 idx_ref, score_ref, loss_ref, load_ref, imp_ref):
    i = pl.program_id(0)
    nsteps = pl.num_programs(0)

    xa = xa_ref[...]
    xb = xb_ref[...]
    w1 = w1_ref[...]
    w2 = w2_ref[...]

    acc = jax.lax.dot_general(
        w1[:, :_HALF], xa, (((1,), (1,)), ((), ())),
        preferred_element_type=jnp.float32)
    acc += jax.lax.dot_general(
        w1[:, _HALF:], xb, (((1,), (1,)), ((), ())),
        preferred_element_type=jnp.float32)
    ht = jnp.tanh(acc)
    logits = jax.lax.dot_general(
        w2, ht, (((1,), (0,)), ((), ())),
        preferred_element_type=jnp.float32)

    m = jnp.max(logits, axis=0, keepdims=True)
    iota = jax.lax.broadcasted_iota(jnp.int32, logits.shape, 0)
    idx = jnp.min(jnp.where(logits == m, iota, _NUM_EXPERTS), axis=0)
    idx_ref[...] = idx
    score_ref[...] = m[0]

    e = jnp.exp(logits - m)
    scores = e / jnp.sum(e, axis=0, keepdims=True)
    imp_part = jnp.sum(scores, axis=1, keepdims=True)

    onehot = (iota == idx[None, :]).astype(jnp.float32)
    load_part = jnp.sum(onehot, axis=1, keepdims=True)

    @pl.when(i == 0)
    def _init():
        imp_ref[...] = jnp.zeros_like(imp_ref)
        load_ref[...] = jnp.zeros_like(load_ref)

    imp_ref[...] += imp_part
    load_ref[...] += load_part

    @pl.when(i == nsteps - 1)
    def _finalize():
        inv_n = 1.0 / _NUM_TOKENS
        imp_mean = imp_ref[...] * inv_n
        load_mean = load_ref[...] * inv_n
        imp_ref[...] = imp_mean
        load_ref[...] = load_mean
        loss_ref[...] = (_NUM_EXPERTS * _BALANCE_LOSS_WEIGHT) * jnp.sum(
            imp_mean * load_mean, axis=0, keepdims=True)


def kernel(x, W1, W2):
    n_tokens = x.shape[0]
    grid = (n_tokens // _BLOCK,)
    idx, score, loss, load_mean, imp_mean = pl.pallas_call(
        _gate_kernel,
        grid=grid,
        in_specs=[
            pl.BlockSpec((_BLOCK, _HALF), lambda i: (i, 0)),
            pl.BlockSpec((_BLOCK, _HALF), lambda i: (i, 1)),
            pl.BlockSpec((_NUM_EXPERTS, _INPUT_SIZE), lambda i: (0, 0)),
            pl.BlockSpec((_NUM_EXPERTS, _NUM_EXPERTS), lambda i: (0, 0)),
        ],
        out_specs=[
            pl.BlockSpec((_BLOCK,), lambda i: (i,)),
            pl.BlockSpec((_BLOCK,), lambda i: (i,)),
            pl.BlockSpec((1, 1), lambda i: (0, 0)),
            pl.BlockSpec((_NUM_EXPERTS, 1), lambda i: (0, 0)),
            pl.BlockSpec((_NUM_EXPERTS, 1), lambda i: (0, 0)),
        ],
        out_shape=[
            jax.ShapeDtypeStruct((n_tokens,), jnp.int32),
            jax.ShapeDtypeStruct((n_tokens,), jnp.float32),
            jax.ShapeDtypeStruct((1, 1), jnp.float32),
            jax.ShapeDtypeStruct((_NUM_EXPERTS, 1), jnp.float32),
            jax.ShapeDtypeStruct((_NUM_EXPERTS, 1), jnp.float32),
        ],
    )(x, x, W1, W2)
    return (idx, score, loss[0, 0], load_mean[:, 0], imp_mean[:, 0])
